# baseline (speedup 1.0000x reference)
import functools

import jax
import jax.numpy as jnp
import numpy as np
from jax import lax
from jax.experimental import pallas as pl
from jax.experimental.pallas import tpu as pltpu
from jax.experimental.pallas import tpu_sc as plsc

ALPHA = 0.05
GAMMA = 0.9
C1 = 1.0 - ALPHA
C2 = 1.0 - ALPHA + ALPHA * GAMMA
GOV = ALPHA * GAMMA / C1

L = 16
B = 64
T = 2048
TW = 2 * T
NWORK = 32
EPW = B // NWORK

_LN1 = float(np.log(C1))
_LN2 = float(np.log(C2))


def _presence_body(la_ref, f0_ref, f1_ref):
    la = la_ref[...]
    prv = pltpu.roll(la, 1, 1)
    odd = lax.iota(jnp.int32, TW) % 2 == 1
    odb = jnp.broadcast_to(odd[None, :], (B, TW))
    p0 = jnp.any(jnp.logical_and(prv >= la, odb))
    p1 = jnp.any(jnp.logical_and(la > prv, odb))
    ones = jnp.ones((8, 128), jnp.float32)
    zero = jnp.zeros((8, 128), jnp.float32)
    f0_ref[...] = jnp.where(p0, ones, zero)
    f1_ref[...] = jnp.where(p1, ones, zero)


def _lane_bcast(x, idx):
    dn = lax.GatherDimensionNumbers(
        offset_dims=(), collapsed_slice_dims=(0,), start_index_map=(0,))
    return lax.gather(x, idx[:, None], dn, slice_sizes=(1,),
                      mode=lax.GatherScatterMode.PROMISE_IN_BOUNDS,
                      indices_are_sorted=True, unique_indices=False)


def _scan_body(r_hbm, f0_hbm, f1_hbm, q_hbm,
               ra_v, rb_v, qa_v, qb_v, f_v,
               sema, semb, semqa, semqb):
    cid = lax.axis_index("c")
    sid = lax.axis_index("s")
    wid = sid * 2 + cid
    epa = wid * EPW
    epb = epa + 1

    cpa = pltpu.async_copy(r_hbm.at[epa], ra_v, sema)
    cpb = pltpu.async_copy(r_hbm.at[epb], rb_v, semb)

    pltpu.sync_copy(f0_hbm.at[0], f_v)
    flag0 = f_v[pl.ds(0, L)] > 0.5
    pltpu.sync_copy(f1_hbm.at[0], f_v)
    flag1 = f_v[pl.ds(0, L)] > 0.5

    ki = lax.iota(jnp.int32, L)
    kf = ki.astype(jnp.float32)
    cn1 = jnp.exp(kf * jnp.float32(-_LN1))
    cp1 = jnp.exp(kf * jnp.float32(_LN1))
    cs1 = cp1 * jnp.float32(C1)
    cn2 = jnp.exp(kf * jnp.float32(-_LN2))
    cp2 = jnp.exp(kf * jnp.float32(_LN2))
    cs2 = cp2 * jnp.float32(C2)
    idx15 = ki * 0 + (L - 1)
    zi = ki * 0
    oi = zi + 1

    cpa.wait()
    cpb.wait()

    def chunk_ep(r_v, q_v, tidx, dc, sc):
        b0 = plsc.load_gather(r_v, [tidx, zi])
        b1 = plsc.load_gather(r_v, [tidx, oi])
        r20 = jnp.where(flag0, 2.0 * b0 - 1.0, b0)
        r21 = jnp.where(flag1, 2.0 * b1 - 1.0, b1)
        bu = ALPHA * (r20 - r21)
        bv = ALPHA * (r20 + r21)
        dch = plsc.cumsum(bu * cn1) * cp1 + dc * cs1
        bw = bv + GOV * jnp.abs(dch - bu)
        sch = plsc.cumsum(bw * cn2) * cp2 + sc * cs2
        plsc.store_scatter(q_v, [tidx, zi], 0.5 * (sch + dch))
        plsc.store_scatter(q_v, [tidx, oi], 0.5 * (sch - dch))
        return _lane_bcast(dch, idx15), _lane_bcast(sch, idx15)

    zeros = jnp.zeros((L,), jnp.float32)
    ones = zeros + 1.0

    @plsc.parallel_loop(0, T // L, 1, unroll=4,
                        carry=(zeros, ones, zeros, ones))
    def chunk(j, carry):
        dca, sca, dcb, scb = carry
        tidx = j * L + ki
        dca, sca = chunk_ep(ra_v, qa_v, tidx, dca, sca)
        dcb, scb = chunk_ep(rb_v, qb_v, tidx, dcb, scb)
        return dca, sca, dcb, scb

    pltpu.async_copy(qa_v, q_hbm.at[epa], semqa)
    cpq = pltpu.async_copy(qb_v, q_hbm.at[epb], semqb)
    pltpu.make_async_copy(qa_v, q_hbm.at[epa], semqa).wait()
    cpq.wait()


_sc_scan = functools.partial(
    pl.kernel,
    out_type=jax.ShapeDtypeStruct((B, T, 2), jnp.float32),
    mesh=plsc.VectorSubcoreMesh(core_axis_name="c", subcore_axis_name="s",
                                num_cores=2, num_subcores=16),
    scratch_types=[
        pltpu.VMEM((T, 2), jnp.float32),
        pltpu.VMEM((T, 2), jnp.float32),
        pltpu.VMEM((T, 2), jnp.float32),
        pltpu.VMEM((T, 2), jnp.float32),
        pltpu.VMEM((128,), jnp.float32),
        pltpu.SemaphoreType.DMA,
        pltpu.SemaphoreType.DMA,
        pltpu.SemaphoreType.DMA,
        pltpu.SemaphoreType.DMA,
    ],
    compiler_params=pltpu.CompilerParams(needs_layout_passes=False,
                                         use_tc_tiling_on_sc=False),
)(_scan_body)


def kernel(state, last_action, rewards):
    del state
    la_flat = last_action.reshape(B, TW)
    f0, f1 = pl.pallas_call(
        _presence_body,
        out_shape=(jax.ShapeDtypeStruct((8, 128), jnp.float32),
                   jax.ShapeDtypeStruct((8, 128), jnp.float32)),
    )(la_flat)
    return _sc_scan(rewards, f0, f1)


# speedup vs baseline: 6.4761x; 6.4761x over previous
import functools

import jax
import jax.numpy as jnp
import numpy as np
from jax import lax
from jax.experimental import pallas as pl
from jax.experimental.pallas import tpu as pltpu
from jax.experimental.pallas import tpu_sc as plsc

ALPHA = 0.05
GAMMA = 0.9
C1 = 1.0 - ALPHA
C2 = 1.0 - ALPHA + ALPHA * GAMMA
GOV = ALPHA * GAMMA / C1

L = 16
B = 64
T = 2048
TW = 2 * T
NWORK = 32
EPW = B // NWORK

_LN1 = float(np.log(C1))
_LN2 = float(np.log(C2))


def _presence_body(la_ref, f0_ref, f1_ref):
    la = la_ref[...]
    prv = pltpu.roll(la, 1, 1)
    odd = lax.iota(jnp.int32, TW) % 2 == 1
    odb = jnp.broadcast_to(odd[None, :], (B, TW))
    p0 = jnp.any(jnp.logical_and(prv >= la, odb))
    p1 = jnp.any(jnp.logical_and(la > prv, odb))
    ones = jnp.ones((8, 128), jnp.float32)
    zero = jnp.zeros((8, 128), jnp.float32)
    f0_ref[...] = jnp.where(p0, ones, zero)
    f1_ref[...] = jnp.where(p1, ones, zero)


def _lane_bcast(x, idx):
    dn = lax.GatherDimensionNumbers(
        offset_dims=(), collapsed_slice_dims=(0,), start_index_map=(0,))
    return lax.gather(x, idx[:, None], dn, slice_sizes=(1,),
                      mode=lax.GatherScatterMode.PROMISE_IN_BOUNDS,
                      indices_are_sorted=True, unique_indices=False)


def _scan_body(r_hbm, f0_hbm, f1_hbm, q_hbm,
               ra_v, rb_v, qa_v, qb_v, f_v,
               sema, semb, semqa, semqb):
    cid = lax.axis_index("c")
    sid = lax.axis_index("s")
    wid = sid * 2 + cid
    epa = wid * EPW
    epb = epa + 1

    cpa = pltpu.async_copy(r_hbm.at[epa], ra_v, sema)
    cpb = pltpu.async_copy(r_hbm.at[epb], rb_v, semb)

    pltpu.sync_copy(f0_hbm.at[0], f_v)
    flag0 = f_v[pl.ds(0, L)] > 0.5
    pltpu.sync_copy(f1_hbm.at[0], f_v)
    flag1 = f_v[pl.ds(0, L)] > 0.5

    ki = lax.iota(jnp.int32, L)
    kf = ki.astype(jnp.float32)
    cn1 = jnp.exp(kf * jnp.float32(-_LN1))
    cp1 = jnp.exp(kf * jnp.float32(_LN1))
    cs1 = cp1 * jnp.float32(C1)
    cn2 = jnp.exp(kf * jnp.float32(-_LN2))
    cp2 = jnp.exp(kf * jnp.float32(_LN2))
    cs2 = cp2 * jnp.float32(C2)
    idx15 = ki * 0 + (L - 1)

    cpa.wait()
    cpb.wait()

    def chunk_ep(r_v, q_v, iev, iod, dc, sc):
        b0 = plsc.load_gather(r_v, [iev])
        b1 = plsc.load_gather(r_v, [iod])
        r20 = jnp.where(flag0, 2.0 * b0 - 1.0, b0)
        r21 = jnp.where(flag1, 2.0 * b1 - 1.0, b1)
        bu = ALPHA * (r20 - r21)
        bv = ALPHA * (r20 + r21)
        dch = plsc.cumsum(bu * cn1) * cp1 + dc * cs1
        bw = bv + GOV * jnp.abs(dch - bu)
        sch = plsc.cumsum(bw * cn2) * cp2 + sc * cs2
        plsc.store_scatter(q_v, [iev], 0.5 * (sch + dch))
        plsc.store_scatter(q_v, [iod], 0.5 * (sch - dch))
        return _lane_bcast(dch, idx15), _lane_bcast(sch, idx15)

    zeros = jnp.zeros((L,), jnp.float32)
    ones = zeros + 1.0
    iev0 = ki * 2
    iod0 = iev0 + 1

    @plsc.parallel_loop(0, T // L, 1, unroll=4,
                        carry=(zeros, ones, zeros, ones))
    def chunk(j, carry):
        dca, sca, dcb, scb = carry
        base = j * (2 * L)
        iev = base + iev0
        iod = base + iod0
        dca, sca = chunk_ep(ra_v, qa_v, iev, iod, dca, sca)
        dcb, scb = chunk_ep(rb_v, qb_v, iev, iod, dcb, scb)
        return dca, sca, dcb, scb

    pltpu.async_copy(qa_v, q_hbm.at[epa], semqa)
    cpq = pltpu.async_copy(qb_v, q_hbm.at[epb], semqb)
    pltpu.make_async_copy(qa_v, q_hbm.at[epa], semqa).wait()
    cpq.wait()


_sc_scan = functools.partial(
    pl.kernel,
    out_type=jax.ShapeDtypeStruct((B, TW), jnp.float32),
    mesh=plsc.VectorSubcoreMesh(core_axis_name="c", subcore_axis_name="s",
                                num_cores=2, num_subcores=16),
    scratch_types=[
        pltpu.VMEM((TW,), jnp.float32),
        pltpu.VMEM((TW,), jnp.float32),
        pltpu.VMEM((TW,), jnp.float32),
        pltpu.VMEM((TW,), jnp.float32),
        pltpu.VMEM((128,), jnp.float32),
        pltpu.SemaphoreType.DMA,
        pltpu.SemaphoreType.DMA,
        pltpu.SemaphoreType.DMA,
        pltpu.SemaphoreType.DMA,
    ],
    compiler_params=pltpu.CompilerParams(needs_layout_passes=False),
)(_scan_body)


def kernel(state, last_action, rewards):
    del state
    la_flat = last_action.reshape(B, TW)
    r_flat = rewards.reshape(B, TW)
    f0, f1 = pl.pallas_call(
        _presence_body,
        out_shape=(jax.ShapeDtypeStruct((8, 128), jnp.float32),
                   jax.ShapeDtypeStruct((8, 128), jnp.float32)),
    )(la_flat)
    q_flat = _sc_scan(r_flat, f0, f1)
    return q_flat.reshape(B, T, 2)
